# Initial kernel scaffold; baseline (speedup 1.0000x reference)
import jax
import jax.numpy as jnp
from jax.experimental import pallas as pl


def kernel(features, distances, cutoffs, rbfs, idx_i, idx_j, params):
    raise NotImplementedError("write your pallas kernel here")


# speedup vs baseline: 2.7740x; 2.7740x over previous
import functools
import math

import jax
import jax.numpy as jnp
from jax import lax
from jax.experimental import pallas as pl
from jax.experimental.pallas import tpu as pltpu
from jax.experimental.pallas import tpu_sc as plsc

N_ATOMS = 10000
N_PAIRS = 320000
F = 128
R = 64
NB = 5
NRI = 3
NRF = 2
_LOG2 = math.log(2.0)

_NC = 2
_NS = 16
_NW = _NC * _NS
_C = 128
_NCHUNK = N_PAIRS // _C
_BASE_CH = _NCHUNK // _NW
_EXTRA = _NCHUNK - _BASE_CH * _NW
_SU = 624
_TAIL = N_ATOMS - _SU * _NS
_ZROWS = 16


def _ssp(v):
    return jnp.maximum(v, 0.0) + jnp.log1p(jnp.exp(-jnp.abs(v))) - _LOG2


_EG_CHUNK = 6400


def _edge_g_body(d_ref, *refs):
    w_refs = refs[:NB]
    g_refs = refs[NB:]
    d = d_ref[...]
    for b in range(NB):
        g_refs[b][...] = jnp.dot(d, w_refs[b][...],
                                 preferred_element_type=jnp.float32)


def _edge_g_call(desc, w_list):
    grid = (N_PAIRS // _EG_CHUNK,)
    return pl.pallas_call(
        _edge_g_body,
        grid=grid,
        in_specs=[pl.BlockSpec((_EG_CHUNK, R), lambda i: (i, 0))] +
                 [pl.BlockSpec((R, F), lambda i: (0, 0))] * NB,
        out_specs=[pl.BlockSpec((_EG_CHUNK, F), lambda i: (i, 0))] * NB,
        out_shape=[jax.ShapeDtypeStruct((N_PAIRS, F), jnp.float32)] * NB,
    )(desc, *w_list)


def _atom_pre_body(x_ref, wi_ref, bi_ref, wj_ref, bj_ref, xi_ref, y_ref):
    xa = _ssp(x_ref[...])
    xi_ref[...] = _ssp(jnp.dot(xa, wi_ref[...],
                               preferred_element_type=jnp.float32) + bi_ref[...])
    y_ref[...] = _ssp(jnp.dot(xa, wj_ref[...],
                              preferred_element_type=jnp.float32) + bj_ref[...])


def _atom_pre_call(x, p):
    return pl.pallas_call(
        _atom_pre_body,
        out_shape=(jax.ShapeDtypeStruct((N_ATOMS, F), jnp.float32),
                   jax.ShapeDtypeStruct((N_ATOMS, F), jnp.float32)),
    )(x, p["dense_i"]["W"], p["dense_i"]["b"].reshape(1, F),
      p["dense_j"]["W"], p["dense_j"]["b"].reshape(1, F))


def _atom_post_body(m01_ref, xi_ref, x_ref, *refs):
    out_ref = refs[-1]
    w = [r[...] for r in refs[:-1]]
    k = 0
    m = m01_ref[0] + m01_ref[1] + xi_ref[...]
    for _ in range(NRI):
        w1, b1, w2, b2 = w[k], w[k + 1], w[k + 2], w[k + 3]
        k += 4
        ym = _ssp(m)
        ym = _ssp(jnp.dot(ym, w1, preferred_element_type=jnp.float32) + b1)
        m = m + jnp.dot(ym, w2, preferred_element_type=jnp.float32) + b2
    wo, bo, u = w[k], w[k + 1], w[k + 2]
    k += 3
    m = _ssp(m)
    x = u * x_ref[...] + jnp.dot(m, wo, preferred_element_type=jnp.float32) + bo
    for _ in range(NRF):
        w1, b1, w2, b2 = w[k], w[k + 1], w[k + 2], w[k + 3]
        k += 4
        yx = _ssp(x)
        yx = _ssp(jnp.dot(yx, w1, preferred_element_type=jnp.float32) + b1)
        x = x + jnp.dot(yx, w2, preferred_element_type=jnp.float32) + b2
    out_ref[...] = x


def _atom_post_call(m01, xi, x, p):
    ws = []
    for rp in p["res_int"]:
        ws += [rp["d1"]["W"], rp["d1"]["b"].reshape(1, F),
               rp["d2"]["W"], rp["d2"]["b"].reshape(1, F)]
    ws += [p["dense_out"]["W"], p["dense_out"]["b"].reshape(1, F),
           p["u"].reshape(1, F)]
    for rp in p["res_feat"]:
        ws += [rp["d1"]["W"], rp["d1"]["b"].reshape(1, F),
               rp["d2"]["W"], rp["d2"]["b"].reshape(1, F)]
    return pl.pallas_call(
        _atom_post_body,
        out_shape=jax.ShapeDtypeStruct((N_ATOMS, F), jnp.float32),
    )(m01, xi, x, *ws)


@functools.cache
def _edge_pass_kernel():
    mesh = plsc.VectorSubcoreMesh(core_axis_name="c", subcore_axis_name="s")

    @functools.partial(
        pl.kernel,
        mesh=mesh,
        out_type=jax.ShapeDtypeStruct((_NC, N_ATOMS, F), jnp.float32),
        scratch_types=[
            pltpu.VMEM((_C,), jnp.int32),
            pltpu.VMEM((_C,), jnp.int32),
            pltpu.VMEM((_C, F), jnp.float32),
            pltpu.VMEM((_C, F), jnp.float32),
            pltpu.VMEM((_ZROWS, F), jnp.float32),
            pltpu.VMEM_SHARED((N_ATOMS, F), jnp.float32),
            pltpu.SemaphoreType.DMA,
        ],
    )
    def _edge_pass(g_hbm, y_hbm, ii_hbm, ij_hbm, out_hbm,
                   ii_v, ij_v, g_v, rows_v, z_v, m_sh, sem):
        return _edge_pass_body(g_hbm, y_hbm, ii_hbm, ij_hbm, out_hbm,
                               ii_v, ij_v, g_v, rows_v, z_v, m_sh, sem)

    return _edge_pass


def _edge_pass_body(g_hbm, y_hbm, ii_hbm, ij_hbm, out_hbm,
                    ii_v, ij_v, g_v, rows_v, z_v, m_sh, sem):
    cid = lax.axis_index("c")
    sid = lax.axis_index("s")
    wid = sid * _NC + cid

    zero = jnp.zeros((16,), jnp.float32)

    def _zrow(i, _):
        for l in range(F // 16):
            z_v[i, pl.ds(l * 16, 16)] = zero
        return 0

    lax.fori_loop(0, _ZROWS, _zrow, 0)

    def _zcopy(i, _):
        pltpu.sync_copy(z_v, m_sh.at[pl.ds(sid * _SU + i * _ZROWS, _ZROWS)])
        return 0

    lax.fori_loop(0, _SU // _ZROWS, _zcopy, 0)

    @pl.when(sid == _NS - 1)
    def _ztail():
        pltpu.sync_copy(z_v, m_sh.at[pl.ds(_SU * _NS, _TAIL)])

    plsc.subcore_barrier()

    def _chunk(t, _):
        base = (t * _NW + wid) * _C
        pltpu.sync_copy(ii_hbm.at[pl.ds(base, _C)], ii_v)
        pltpu.sync_copy(ij_hbm.at[pl.ds(base, _C)], ij_v)
        pltpu.sync_copy(g_hbm.at[pl.ds(base, _C)], g_v)
        pltpu.async_copy(y_hbm.at[ij_v], rows_v, sem).wait()

        def _mul(i, _):
            for l in range(F // 16):
                s = pl.ds(l * 16, 16)
                rows_v[i, s] = rows_v[i, s] * g_v[i, s]
            return 0

        lax.fori_loop(0, _C, _mul, 0)
        pltpu.sync_copy(rows_v, m_sh.at[ii_v], add=True)
        return 0

    nch = _BASE_CH + jnp.where(wid < _EXTRA, 1, 0)
    lax.fori_loop(0, nch, _chunk, 0)
    plsc.subcore_barrier()

    pltpu.sync_copy(m_sh.at[pl.ds(sid * _SU, _SU)],
                    out_hbm.at[cid, pl.ds(sid * _SU, _SU)])

    @pl.when(sid == _NS - 1)
    def _wtail():
        pltpu.sync_copy(m_sh.at[pl.ds(_SU * _NS, _TAIL)],
                        out_hbm.at[cid, pl.ds(_SU * _NS, _TAIL)])


def kernel(features, distances, cutoffs, rbfs, idx_i, idx_j, params):
    del distances
    desc = cutoffs[:, None] * rbfs
    blocks = params["blocks"]
    gs = _edge_g_call(desc, [p["W_desc"] for p in blocks])
    x = features
    outs = []
    for b in range(NB):
        p = blocks[b]
        xi, y = _atom_pre_call(x, p)
        m01 = _edge_pass_kernel()(gs[b], y, idx_i, idx_j)
        x = _atom_post_call(m01, xi, x, p)
        outs.append(x)
    return jnp.stack(outs, axis=0)


# speedup vs baseline: 4.3154x; 1.5556x over previous
import functools
import math

import jax
import jax.numpy as jnp
from jax import lax
from jax.experimental import pallas as pl
from jax.experimental.pallas import tpu as pltpu
from jax.experimental.pallas import tpu_sc as plsc

N_ATOMS = 10000
N_PAIRS = 320000
F = 128
R = 64
NB = 5
NRI = 3
NRF = 2
_LOG2 = math.log(2.0)

_NC = 2
_NS = 16
_NW = _NC * _NS
_C = 64
_EW = N_PAIRS // _NW
_FULL_CH = _EW // _C
_ETAIL = _EW - _FULL_CH * _C
_SU = 624
_TAIL = N_ATOMS - _SU * _NS


def _ssp(v):
    return jnp.maximum(v, 0.0) + jnp.log1p(jnp.exp(-jnp.abs(v))) - _LOG2


_EG_CHUNK = 6400


def _edge_g_body(d_ref, *refs):
    w_refs = refs[:NB]
    g_refs = refs[NB:]
    d = d_ref[...]
    for b in range(NB):
        g_refs[b][...] = jnp.dot(d, w_refs[b][...],
                                 preferred_element_type=jnp.float32)


def _edge_g_call(desc, w_list):
    grid = (N_PAIRS // _EG_CHUNK,)
    return pl.pallas_call(
        _edge_g_body,
        grid=grid,
        in_specs=[pl.BlockSpec((_EG_CHUNK, R), lambda i: (i, 0))] +
                 [pl.BlockSpec((R, F), lambda i: (0, 0))] * NB,
        out_specs=[pl.BlockSpec((_EG_CHUNK, F), lambda i: (i, 0))] * NB,
        out_shape=[jax.ShapeDtypeStruct((N_PAIRS, F), jnp.float32)] * NB,
    )(desc, *w_list)


def _atom_pre_body(x_ref, wi_ref, bi_ref, wj_ref, bj_ref, xi_ref, y_ref):
    xa = _ssp(x_ref[...])
    xi_ref[...] = _ssp(jnp.dot(xa, wi_ref[...],
                               preferred_element_type=jnp.float32) + bi_ref[...])
    y_ref[...] = _ssp(jnp.dot(xa, wj_ref[...],
                              preferred_element_type=jnp.float32) + bj_ref[...])


def _atom_pre_call(x, p):
    return pl.pallas_call(
        _atom_pre_body,
        out_shape=(jax.ShapeDtypeStruct((N_ATOMS, F), jnp.float32),
                   jax.ShapeDtypeStruct((N_ATOMS, F), jnp.float32)),
    )(x, p["dense_i"]["W"], p["dense_i"]["b"].reshape(1, F),
      p["dense_j"]["W"], p["dense_j"]["b"].reshape(1, F))


def _atom_post_body(m01_ref, xi_ref, x_ref, *refs):
    out_ref = refs[-1]
    w = [r[...] for r in refs[:-1]]
    k = 0
    m = m01_ref[0] + m01_ref[1] + xi_ref[...]
    for _ in range(NRI):
        w1, b1, w2, b2 = w[k], w[k + 1], w[k + 2], w[k + 3]
        k += 4
        ym = _ssp(m)
        ym = _ssp(jnp.dot(ym, w1, preferred_element_type=jnp.float32) + b1)
        m = m + jnp.dot(ym, w2, preferred_element_type=jnp.float32) + b2
    wo, bo, u = w[k], w[k + 1], w[k + 2]
    k += 3
    m = _ssp(m)
    x = u * x_ref[...] + jnp.dot(m, wo, preferred_element_type=jnp.float32) + bo
    for _ in range(NRF):
        w1, b1, w2, b2 = w[k], w[k + 1], w[k + 2], w[k + 3]
        k += 4
        yx = _ssp(x)
        yx = _ssp(jnp.dot(yx, w1, preferred_element_type=jnp.float32) + b1)
        x = x + jnp.dot(yx, w2, preferred_element_type=jnp.float32) + b2
    out_ref[...] = x


def _atom_post_call(m01, xi, x, p):
    ws = []
    for rp in p["res_int"]:
        ws += [rp["d1"]["W"], rp["d1"]["b"].reshape(1, F),
               rp["d2"]["W"], rp["d2"]["b"].reshape(1, F)]
    ws += [p["dense_out"]["W"], p["dense_out"]["b"].reshape(1, F),
           p["u"].reshape(1, F)]
    for rp in p["res_feat"]:
        ws += [rp["d1"]["W"], rp["d1"]["b"].reshape(1, F),
               rp["d2"]["W"], rp["d2"]["b"].reshape(1, F)]
    return pl.pallas_call(
        _atom_post_body,
        out_shape=jax.ShapeDtypeStruct((N_ATOMS, F), jnp.float32),
    )(m01, xi, x, *ws)


@functools.cache
def _edge_pass_kernel():
    mesh = plsc.VectorSubcoreMesh(core_axis_name="c", subcore_axis_name="s")

    @functools.partial(
        pl.kernel,
        mesh=mesh,
        out_type=jax.ShapeDtypeStruct((_NC, N_ATOMS, F), jnp.float32),
        scratch_types=[
            [pltpu.VMEM((_C,), jnp.int32)] * 4,
            [pltpu.VMEM((_C,), jnp.int32)] * 4,
            [pltpu.VMEM((_C, F), jnp.float32)] * 2,
            [pltpu.VMEM((_C, F), jnp.float32)] * 2,
            pltpu.VMEM((_ETAIL,), jnp.int32),
            pltpu.VMEM((_ETAIL,), jnp.int32),
            pltpu.VMEM((_ETAIL, F), jnp.float32),
            pltpu.VMEM((_ETAIL, F), jnp.float32),
            pltpu.VMEM_SHARED((N_ATOMS, F), jnp.float32),
            [pltpu.SemaphoreType.DMA] * 4,
            [pltpu.SemaphoreType.DMA] * 2,
            [pltpu.SemaphoreType.DMA] * 2,
            [pltpu.SemaphoreType.DMA] * 2,
            pltpu.SemaphoreType.DMA,
            pltpu.SemaphoreType.DMA,
            pltpu.SemaphoreType.DMA,
        ],
    )
    def _edge_pass(g_hbm, y_hbm, ii_hbm, ij_hbm, out_hbm,
                   ii_c, ij_c, g_v, rows_v, ii_t, ij_t, g_t, rows_t, m_sh,
                   sem_i, sem_g, sem_r, sem_s, sem_it, sem_gt, sem_rt):
        return _edge_pass_body(g_hbm, y_hbm, ii_hbm, ij_hbm, out_hbm,
                               ii_c, ij_c, g_v, rows_v,
                               ii_t, ij_t, g_t, rows_t, m_sh,
                               sem_i, sem_g, sem_r, sem_s,
                               sem_it, sem_gt, sem_rt)

    return _edge_pass


def _edge_pass_body(g_hbm, y_hbm, ii_hbm, ij_hbm, out_hbm,
                    ii_c, ij_c, g_v, rows_v, ii_t, ij_t, g_t, rows_t, m_sh,
                    sem_i, sem_g, sem_r, sem_s, sem_it, sem_gt, sem_rt):
    cid = lax.axis_index("c")
    sid = lax.axis_index("s")
    wid = cid * _NS + sid
    ebase = wid * _EW

    def _issue_idx(u, s):
        pltpu.async_copy(ii_hbm.at[pl.ds(ebase + u * _C, _C)], ii_c[s],
                         sem_i[s])
        pltpu.async_copy(ij_hbm.at[pl.ds(ebase + u * _C, _C)], ij_c[s],
                         sem_i[s])

    def _wait_idx(s):
        pltpu.make_async_copy(ii_hbm.at[pl.ds(0, _C)], ii_c[s],
                              sem_i[s]).wait()
        pltpu.make_async_copy(ij_hbm.at[pl.ds(0, _C)], ij_c[s],
                              sem_i[s]).wait()

    def _issue_data(u, b, s):
        pltpu.async_copy(g_hbm.at[pl.ds(ebase + u * _C, _C)], g_v[b],
                         sem_g[b])
        pltpu.async_copy(y_hbm.at[ij_c[s]], rows_v[b], sem_r[b])

    for s in range(3):
        _issue_idx(s, s)
    pltpu.async_copy(ii_hbm.at[pl.ds(ebase + _FULL_CH * _C, _ETAIL)], ii_t,
                     sem_it)
    pltpu.async_copy(ij_hbm.at[pl.ds(ebase + _FULL_CH * _C, _ETAIL)], ij_t,
                     sem_it)

    zero = jnp.zeros((16,), jnp.float32)

    def _zrow(i, _):
        for b in range(2):
            for l in range(F // 16):
                rows_v[b][i, pl.ds(l * 16, 16)] = zero
        return 0

    lax.fori_loop(0, _C, _zrow, 0)
    for r in range(9):
        pltpu.sync_copy(rows_v[r % 2],
                        m_sh.at[pl.ds(sid * _SU + r * _C, _C)])
    pltpu.sync_copy(rows_v[1].at[pl.ds(0, _SU - 9 * _C)],
                    m_sh.at[pl.ds(sid * _SU + 9 * _C, _SU - 9 * _C)])

    @pl.when(sid == _NS - 1)
    def _ztail():
        pltpu.sync_copy(rows_v[0].at[pl.ds(0, _TAIL)],
                        m_sh.at[pl.ds(_SU * _NS, _TAIL)])

    pltpu.make_async_copy(ii_hbm.at[pl.ds(0, _ETAIL)], ii_t, sem_it).wait()
    pltpu.make_async_copy(ij_hbm.at[pl.ds(0, _ETAIL)], ij_t, sem_it).wait()
    pltpu.async_copy(g_hbm.at[pl.ds(ebase + _FULL_CH * _C, _ETAIL)], g_t,
                     sem_gt)
    pltpu.async_copy(y_hbm.at[ij_t], rows_t, sem_rt)
    _wait_idx(0)
    _issue_data(0, 0, 0)

    plsc.subcore_barrier()

    def _step(k, b4):
        u = 4 * k + b4
        b = b4 % 2
        bo = 1 - b
        pltpu.make_async_copy(g_hbm.at[pl.ds(0, _C)], g_v[b],
                              sem_g[b]).wait()
        pltpu.make_async_copy(y_hbm.at[ij_c[b4]], rows_v[b],
                              sem_r[b]).wait()

        def _retire():
            pltpu.make_async_copy(rows_v[bo], m_sh.at[ii_c[(b4 + 3) % 4]],
                                  sem_s[bo]).wait()

        if b4 == 0:
            pl.when(k > 0)(_retire)
        else:
            _retire()

        def _launch_next():
            _wait_idx((b4 + 1) % 4)
            _issue_data(u + 1, bo, (b4 + 1) % 4)

        if b4 == 3:
            pl.when(k < _FULL_CH // 4 - 1)(_launch_next)
        else:
            _launch_next()

        def _prefetch_idx():
            _issue_idx(u + 3, (b4 + 3) % 4)

        if b4 == 0:
            _prefetch_idx()
        else:
            pl.when(k < _FULL_CH // 4 - 1)(_prefetch_idx)

        @plsc.parallel_loop(0, _C, 1, unroll=2)
        def _mul(i):
            for l in range(F // 16):
                s = pl.ds(l * 16, 16)
                rows_v[b][i, s] = rows_v[b][i, s] * g_v[b][i, s]

        pltpu.async_copy(rows_v[b], m_sh.at[ii_c[b4]], sem_s[b], add=True)

    def _quad(k, _):
        for b4 in range(4):
            _step(k, b4)
        return 0

    lax.fori_loop(0, _FULL_CH // 4, _quad, 0)

    pltpu.make_async_copy(rows_v[1], m_sh.at[ii_c[3]], sem_s[1]).wait()
    pltpu.make_async_copy(g_hbm.at[pl.ds(0, _ETAIL)], g_t, sem_gt).wait()
    pltpu.make_async_copy(y_hbm.at[ij_t], rows_t, sem_rt).wait()

    def _mul_t(i, _):
        for l in range(F // 16):
            s = pl.ds(l * 16, 16)
            rows_t[i, s] = rows_t[i, s] * g_t[i, s]
        return 0

    lax.fori_loop(0, _ETAIL, _mul_t, 0)
    pltpu.sync_copy(rows_t, m_sh.at[ii_t], add=True)

    plsc.subcore_barrier()

    pltpu.sync_copy(m_sh.at[pl.ds(sid * _SU, _SU)],
                    out_hbm.at[cid, pl.ds(sid * _SU, _SU)])

    @pl.when(sid == _NS - 1)
    def _wtail():
        pltpu.sync_copy(m_sh.at[pl.ds(_SU * _NS, _TAIL)],
                        out_hbm.at[cid, pl.ds(_SU * _NS, _TAIL)])


def kernel(features, distances, cutoffs, rbfs, idx_i, idx_j, params):
    del distances
    desc = cutoffs[:, None] * rbfs
    blocks = params["blocks"]
    gs = _edge_g_call(desc, [p["W_desc"] for p in blocks])
    x = features
    outs = []
    for b in range(NB):
        p = blocks[b]
        xi, y = _atom_pre_call(x, p)
        m01 = _edge_pass_kernel()(gs[b], y, idx_i, idx_j)
        x = _atom_post_call(m01, xi, x, p)
        outs.append(x)
    return jnp.stack(outs, axis=0)
